# Initial kernel scaffold; baseline (speedup 1.0000x reference)
import jax
import jax.numpy as jnp
from jax.experimental import pallas as pl


def kernel(xy, tables):
    raise NotImplementedError("write your pallas kernel here")


# speedup vs baseline: 122.8488x; 122.8488x over previous
import jax
import jax.numpy as jnp
from jax import lax
from jax.experimental import pallas as pl
from jax.experimental.pallas import tpu as pltpu
from jax.experimental.pallas import tpu_sc as plsc

L = 16
T = 524288
F = 2
GROWTH = 1.38
PI1 = -1640531535
PI2 = 805459861
MASK = T - 1
RES = [int(16 * GROWTH ** lev) for lev in range(L)]

NCORES = 2
NSUB = 16
NW = NCORES * NSUB
LANES = 16
C = 1024


def _body(xs_hbm, ys_hbm, tab_hbm, out_hbm,
          u_v, v_v, fx_v, fy_v, idx_v, rows_v, out_v, sem):
    n = xs_hbm.shape[0]
    ppw = n // NW
    nchunk = ppw // C
    cid = lax.axis_index("c")
    sid = lax.axis_index("s")
    wid = sid * NCORES + cid

    iota = lax.iota(jnp.int32, LANES)
    iota32 = iota * (2 * L)

    @pl.loop(0, nchunk)
    def _chunk(ci):
        base = wid * ppw + ci * C
        pltpu.sync_copy(xs_hbm.at[pl.ds(base, C)], u_v)
        pltpu.sync_copy(ys_hbm.at[pl.ds(base, C)], v_v)

        @pl.loop(0, C // LANES)
        def _pre(i):
            sl = pl.ds(i * LANES, LANES)
            u_v[sl] = u_v[sl] * 0.5 + 0.5
            v_v[sl] = v_v[sl] * 0.5 + 0.5

        for lev in range(L):
            resf = float(RES[lev])
            lev_base = lev * T

            @pl.loop(0, C // LANES)
            def _hash(i):
                sl = pl.ds(i * LANES, LANES)
                sx = u_v[sl] * resf
                sy = v_v[sl] * resf
                ix = sx.astype(jnp.int32)
                iy = sy.astype(jnp.int32)
                fx_v[sl] = sx - ix.astype(jnp.float32)
                fy_v[sl] = sy - iy.astype(jnp.float32)
                a0 = ix * PI1
                a1 = a0 + PI1
                b0 = iy * PI2
                b1 = b0 + PI2
                idx_v[pl.ds(0 * C + i * LANES, LANES)] = ((a0 ^ b0) & MASK) + lev_base
                idx_v[pl.ds(1 * C + i * LANES, LANES)] = ((a1 ^ b0) & MASK) + lev_base
                idx_v[pl.ds(2 * C + i * LANES, LANES)] = ((a0 ^ b1) & MASK) + lev_base
                idx_v[pl.ds(3 * C + i * LANES, LANES)] = ((a1 ^ b1) & MASK) + lev_base

            pltpu.async_copy(tab_hbm.at[idx_v], rows_v, sem).wait()

            @pl.loop(0, C // LANES)
            def _comb(i):
                sl = pl.ds(i * LANES, LANES)
                fx = fx_v[sl]
                fy = fy_v[sl]
                w00 = rows_v[pl.ds(0 * C + i * LANES, LANES)]
                w10 = rows_v[pl.ds(1 * C + i * LANES, LANES)]
                w01 = rows_v[pl.ds(2 * C + i * LANES, LANES)]
                w11 = rows_v[pl.ds(3 * C + i * LANES, LANES)]
                hi = jnp.int32(-65536)
                c00_0 = plsc.bitcast(w00 << 16, jnp.float32)
                c10_0 = plsc.bitcast(w10 << 16, jnp.float32)
                c01_0 = plsc.bitcast(w01 << 16, jnp.float32)
                c11_0 = plsc.bitcast(w11 << 16, jnp.float32)
                c00_1 = plsc.bitcast(w00 & hi, jnp.float32)
                c10_1 = plsc.bitcast(w10 & hi, jnp.float32)
                c01_1 = plsc.bitcast(w01 & hi, jnp.float32)
                c11_1 = plsc.bitcast(w11 & hi, jnp.float32)
                t0 = c00_0 + fx * (c10_0 - c00_0)
                t1 = c01_0 + fx * (c11_0 - c01_0)
                f0 = t0 + fy * (t1 - t0)
                s0 = c00_1 + fx * (c10_1 - c00_1)
                s1 = c01_1 + fx * (c11_1 - c01_1)
                f1 = s0 + fy * (s1 - s0)
                ov = i * (LANES * 2 * L) + iota32 + (2 * lev)
                plsc.store_scatter(out_v, [ov], f0)
                plsc.store_scatter(out_v, [ov + 1], f1)

        pltpu.sync_copy(out_v, out_hbm.at[pl.ds(base * (2 * L), C * 2 * L)])


def kernel(xy, tables):
    n = xy.shape[0]
    xs = xy[:, 0]
    ys = xy[:, 1]
    tb = tables.astype(jnp.bfloat16).reshape(L * T, F)
    tabw = lax.bitcast_convert_type(
        lax.bitcast_convert_type(tb, jnp.uint16), jnp.int32
    ).reshape(L * T)
    mesh = plsc.VectorSubcoreMesh(core_axis_name="c", subcore_axis_name="s")
    run = pl.kernel(
        _body,
        out_type=jax.ShapeDtypeStruct((n * 2 * L,), jnp.float32),
        mesh=mesh,
        compiler_params=pltpu.CompilerParams(needs_layout_passes=False),
        scratch_types=[
            pltpu.VMEM((C,), jnp.float32),
            pltpu.VMEM((C,), jnp.float32),
            pltpu.VMEM((C,), jnp.float32),
            pltpu.VMEM((C,), jnp.float32),
            pltpu.VMEM((4 * C,), jnp.int32),
            pltpu.VMEM((4 * C,), jnp.int32),
            pltpu.VMEM((C * 2 * L,), jnp.float32),
            pltpu.SemaphoreType.DMA,
        ],
    )
    flat = run(xs, ys, tabw)
    return flat.reshape(n, 2 * L)


# speedup vs baseline: 188.1694x; 1.5317x over previous
import jax
import jax.numpy as jnp
from jax import lax
from jax.experimental import pallas as pl
from jax.experimental.pallas import tpu as pltpu
from jax.experimental.pallas import tpu_sc as plsc

L = 16
T = 524288
F = 2
GROWTH = 1.38
PI1 = -1640531535
PI2 = 805459861
MASK = T - 1
RES = [int(16 * GROWTH ** lev) for lev in range(L)]

NCORES = 2
NSUB = 16
NW = NCORES * NSUB
LANES = 16
C = 1024


def _body(xs_hbm, ys_hbm, tab_hbm, out_hbm,
          xs_v, ys_v, idx0, idx1, rows0, rows1,
          fx0, fx1, fy0, fy1, out_v, sem0, sem1):
    n = xs_hbm.shape[0]
    ppw = n // NW
    nchunk = ppw // C
    cid = lax.axis_index("c")
    sid = lax.axis_index("s")
    wid = sid * NCORES + cid

    iota = lax.iota(jnp.int32, LANES)
    iota32 = iota * (2 * L)
    bufs = ((idx0, rows0, fx0, fy0, sem0), (idx1, rows1, fx1, fy1, sem1))

    def hash_level(lev, idx_v, fx_v, fy_v):
        hr = float(RES[lev]) * 0.5
        lev_base = lev * T

        @pl.loop(0, C // LANES)
        def _hash(i):
            sl = pl.ds(i * LANES, LANES)
            sx = xs_v[sl] * hr + hr
            sy = ys_v[sl] * hr + hr
            ix = sx.astype(jnp.int32)
            iy = sy.astype(jnp.int32)
            fx_v[sl] = sx - ix.astype(jnp.float32)
            fy_v[sl] = sy - iy.astype(jnp.float32)
            a0 = ix * PI1
            a1 = a0 + PI1
            b0 = iy * PI2
            b1 = b0 + PI2
            idx_v[pl.ds(0 * C + i * LANES, LANES)] = ((a0 ^ b0) & MASK) + lev_base
            idx_v[pl.ds(1 * C + i * LANES, LANES)] = ((a1 ^ b0) & MASK) + lev_base
            idx_v[pl.ds(2 * C + i * LANES, LANES)] = ((a0 ^ b1) & MASK) + lev_base
            idx_v[pl.ds(3 * C + i * LANES, LANES)] = ((a1 ^ b1) & MASK) + lev_base

    def combine_level(lev, rows_v, fx_v, fy_v):
        @pl.loop(0, C // LANES)
        def _comb(i):
            sl = pl.ds(i * LANES, LANES)
            fx = fx_v[sl]
            fy = fy_v[sl]
            w00 = rows_v[pl.ds(0 * C + i * LANES, LANES)]
            w10 = rows_v[pl.ds(1 * C + i * LANES, LANES)]
            w01 = rows_v[pl.ds(2 * C + i * LANES, LANES)]
            w11 = rows_v[pl.ds(3 * C + i * LANES, LANES)]
            hi = jnp.int32(-65536)
            c00_0 = plsc.bitcast(w00 << 16, jnp.float32)
            c10_0 = plsc.bitcast(w10 << 16, jnp.float32)
            c01_0 = plsc.bitcast(w01 << 16, jnp.float32)
            c11_0 = plsc.bitcast(w11 << 16, jnp.float32)
            c00_1 = plsc.bitcast(w00 & hi, jnp.float32)
            c10_1 = plsc.bitcast(w10 & hi, jnp.float32)
            c01_1 = plsc.bitcast(w01 & hi, jnp.float32)
            c11_1 = plsc.bitcast(w11 & hi, jnp.float32)
            t0 = c00_0 + fx * (c10_0 - c00_0)
            t1 = c01_0 + fx * (c11_0 - c01_0)
            f0 = t0 + fy * (t1 - t0)
            s0 = c00_1 + fx * (c10_1 - c00_1)
            s1 = c01_1 + fx * (c11_1 - c01_1)
            f1 = s0 + fy * (s1 - s0)
            ov = i * (LANES * 2 * L) + iota32 + (2 * lev)
            plsc.store_scatter(out_v, [ov], f0)
            plsc.store_scatter(out_v, [ov + 1], f1)

    @pl.loop(0, nchunk)
    def _chunk(ci):
        base = wid * ppw + ci * C
        pltpu.sync_copy(xs_hbm.at[pl.ds(base, C)], xs_v)
        pltpu.sync_copy(ys_hbm.at[pl.ds(base, C)], ys_v)

        idx_v, rows_v, fx_v, fy_v, sem = bufs[0]
        hash_level(0, idx_v, fx_v, fy_v)
        cps = [pltpu.async_copy(tab_hbm.at[idx_v], rows_v, sem)]
        for lev in range(1, L):
            idx_v, rows_v, fx_v, fy_v, sem = bufs[lev & 1]
            hash_level(lev, idx_v, fx_v, fy_v)
            cps.append(pltpu.async_copy(tab_hbm.at[idx_v], rows_v, sem))
            p_idx, p_rows, p_fx, p_fy, _ = bufs[(lev - 1) & 1]
            cps[lev - 1].wait()
            combine_level(lev - 1, p_rows, p_fx, p_fy)
        p_idx, p_rows, p_fx, p_fy, _ = bufs[(L - 1) & 1]
        cps[L - 1].wait()
        combine_level(L - 1, p_rows, p_fx, p_fy)

        pltpu.sync_copy(out_v, out_hbm.at[pl.ds(base * (2 * L), C * 2 * L)])


def kernel(xy, tables):
    n = xy.shape[0]
    xs = xy[:, 0]
    ys = xy[:, 1]
    tb = tables.astype(jnp.bfloat16).reshape(L * T, F)
    tabw = lax.bitcast_convert_type(
        lax.bitcast_convert_type(tb, jnp.uint16), jnp.int32
    ).reshape(L * T)
    mesh = plsc.VectorSubcoreMesh(core_axis_name="c", subcore_axis_name="s")
    run = pl.kernel(
        _body,
        out_type=jax.ShapeDtypeStruct((n * 2 * L,), jnp.float32),
        mesh=mesh,
        compiler_params=pltpu.CompilerParams(needs_layout_passes=False),
        scratch_types=[
            pltpu.VMEM((C,), jnp.float32),
            pltpu.VMEM((C,), jnp.float32),
            pltpu.VMEM((4 * C,), jnp.int32),
            pltpu.VMEM((4 * C,), jnp.int32),
            pltpu.VMEM((4 * C,), jnp.int32),
            pltpu.VMEM((4 * C,), jnp.int32),
            pltpu.VMEM((C,), jnp.float32),
            pltpu.VMEM((C,), jnp.float32),
            pltpu.VMEM((C,), jnp.float32),
            pltpu.VMEM((C,), jnp.float32),
            pltpu.VMEM((C * 2 * L,), jnp.float32),
            pltpu.SemaphoreType.DMA,
            pltpu.SemaphoreType.DMA,
        ],
    )
    flat = run(xs, ys, tabw)
    return flat.reshape(n, 2 * L)


# speedup vs baseline: 237.6119x; 1.2628x over previous
import numpy as np

import jax
import jax.numpy as jnp
from jax import lax
from jax.experimental import pallas as pl
from jax.experimental.pallas import tpu as pltpu
from jax.experimental.pallas import tpu_sc as plsc

L = 16
T = 524288
F = 2
GROWTH = 1.38
PI1 = -1640531535
PI2 = 805459861
MASK = T - 1
RES = [int(16 * GROWTH ** lev) for lev in range(L)]

NCORES = 2
NSUB = 16
NW = NCORES * NSUB
LANES = 16
C = 1024

NCOARSE = 8
WP = [((RES[lev] + 1 + LANES - 1) // LANES) * LANES for lev in range(NCOARSE)]
GBASE = [0]
for lev in range(NCOARSE):
    GBASE.append(GBASE[-1] + (RES[lev] + 1) * WP[lev])
GTOT = GBASE[NCOARSE]
SEG = 4 * C
GPAD = ((GTOT + SEG - 1) // SEG) * SEG


def _build_gidx() -> np.ndarray:
    out = np.zeros(GPAD, np.int32)
    for lev in range(NCOARSE):
        w = RES[lev] + 1
        gx = np.arange(WP[lev], dtype=np.int64)
        gy = np.arange(w, dtype=np.int64)
        ax = ((gx * PI1) & 0xFFFFFFFF).astype(np.uint32)
        by = ((gy * PI2) & 0xFFFFFFFF).astype(np.uint32)
        h = ((by[:, None] ^ ax[None, :]) & np.uint32(MASK)).astype(np.int64)
        out[GBASE[lev]:GBASE[lev + 1]] = (h + lev * T).astype(np.int32).ravel()
    return out


GIDX = _build_gidx()


def _body(xs_hbm, ys_hbm, tab_hbm, gidx_hbm, out_hbm,
          xs_v, ys_v, grid_v, idx0, idx1, rows0, rows1,
          fx0, fx1, fy0, fy1, out_v, sem0, sem1):
    n = xs_hbm.shape[0]
    ppw = n // NW
    nchunk = ppw // C
    cid = lax.axis_index("c")
    sid = lax.axis_index("s")
    wid = sid * NCORES + cid

    iota = lax.iota(jnp.int32, LANES)
    iota32 = iota * (2 * L)
    hi = jnp.int32(-65536)
    bufs = ((idx0, rows0, fx0, fy0, sem0), (idx1, rows1, fx1, fy1, sem1))

    bcps = []
    for s in range(GPAD // SEG):
        idx_v, _, _, _, sem = bufs[s & 1]
        if s >= 2:
            bcps[s - 2].wait()
        pltpu.sync_copy(gidx_hbm.at[pl.ds(s * SEG, SEG)], idx_v)
        bcps.append(
            pltpu.async_copy(tab_hbm.at[idx_v], grid_v.at[pl.ds(s * SEG, SEG)], sem))
    bcps[-2].wait()
    bcps[-1].wait()

    def lerp2(w00, w10, w01, w11, fx, fy, lev, i):
        c00_0 = plsc.bitcast(w00 << 16, jnp.float32)
        c10_0 = plsc.bitcast(w10 << 16, jnp.float32)
        c01_0 = plsc.bitcast(w01 << 16, jnp.float32)
        c11_0 = plsc.bitcast(w11 << 16, jnp.float32)
        c00_1 = plsc.bitcast(w00 & hi, jnp.float32)
        c10_1 = plsc.bitcast(w10 & hi, jnp.float32)
        c01_1 = plsc.bitcast(w01 & hi, jnp.float32)
        c11_1 = plsc.bitcast(w11 & hi, jnp.float32)
        t0 = c00_0 + fx * (c10_0 - c00_0)
        t1 = c01_0 + fx * (c11_0 - c01_0)
        f0 = t0 + fy * (t1 - t0)
        s0 = c00_1 + fx * (c10_1 - c00_1)
        s1 = c01_1 + fx * (c11_1 - c01_1)
        f1 = s0 + fy * (s1 - s0)
        ov = i * (LANES * 2 * L) + iota32 + (2 * lev)
        plsc.store_scatter(out_v, [ov], f0)
        plsc.store_scatter(out_v, [ov + 1], f1)

    def coarse_level(lev):
        hr = float(RES[lev]) * 0.5
        wp = WP[lev]
        gb = GBASE[lev]

        @pl.loop(0, C // LANES)
        def _cl(i):
            sl = pl.ds(i * LANES, LANES)
            sx = xs_v[sl] * hr + hr
            sy = ys_v[sl] * hr + hr
            ix = sx.astype(jnp.int32)
            iy = sy.astype(jnp.int32)
            fx = sx - ix.astype(jnp.float32)
            fy = sy - iy.astype(jnp.float32)
            g = iy * wp + ix + gb
            w00 = plsc.load_gather(grid_v, [g])
            w10 = plsc.load_gather(grid_v, [g + 1])
            w01 = plsc.load_gather(grid_v, [g + wp])
            w11 = plsc.load_gather(grid_v, [g + (wp + 1)])
            lerp2(w00, w10, w01, w11, fx, fy, lev, i)

    def hash_fine(lev, idx_v, fx_v, fy_v):
        hr = float(RES[lev]) * 0.5
        lev_base = lev * T

        @pl.loop(0, C // LANES)
        def _hash(i):
            sl = pl.ds(i * LANES, LANES)
            sx = xs_v[sl] * hr + hr
            sy = ys_v[sl] * hr + hr
            ix = sx.astype(jnp.int32)
            iy = sy.astype(jnp.int32)
            fx_v[sl] = sx - ix.astype(jnp.float32)
            fy_v[sl] = sy - iy.astype(jnp.float32)
            a0 = ix * PI1
            a1 = a0 + PI1
            b0 = iy * PI2
            b1 = b0 + PI2
            idx_v[pl.ds(0 * C + i * LANES, LANES)] = ((a0 ^ b0) & MASK) + lev_base
            idx_v[pl.ds(1 * C + i * LANES, LANES)] = ((a1 ^ b0) & MASK) + lev_base
            idx_v[pl.ds(2 * C + i * LANES, LANES)] = ((a0 ^ b1) & MASK) + lev_base
            idx_v[pl.ds(3 * C + i * LANES, LANES)] = ((a1 ^ b1) & MASK) + lev_base

    def combine_fine(lev, rows_v, fx_v, fy_v):
        @pl.loop(0, C // LANES)
        def _comb(i):
            sl = pl.ds(i * LANES, LANES)
            fx = fx_v[sl]
            fy = fy_v[sl]
            w00 = rows_v[pl.ds(0 * C + i * LANES, LANES)]
            w10 = rows_v[pl.ds(1 * C + i * LANES, LANES)]
            w01 = rows_v[pl.ds(2 * C + i * LANES, LANES)]
            w11 = rows_v[pl.ds(3 * C + i * LANES, LANES)]
            lerp2(w00, w10, w01, w11, fx, fy, lev, i)

    @pl.loop(0, nchunk)
    def _chunk(ci):
        base = wid * ppw + ci * C
        pltpu.sync_copy(xs_hbm.at[pl.ds(base, C)], xs_v)
        pltpu.sync_copy(ys_hbm.at[pl.ds(base, C)], ys_v)

        cps = {}
        for lev in (NCOARSE, NCOARSE + 1):
            idx_v, rows_v, fx_v, fy_v, sem = bufs[lev & 1]
            hash_fine(lev, idx_v, fx_v, fy_v)
            cps[lev] = pltpu.async_copy(tab_hbm.at[idx_v], rows_v, sem)

        for lev in range(NCOARSE):
            coarse_level(lev)

        for lev in range(NCOARSE + 2, L):
            idx_v, rows_v, fx_v, fy_v, sem = bufs[lev & 1]
            cps[lev - 2].wait()
            combine_fine(lev - 2, rows_v, fx_v, fy_v)
            hash_fine(lev, idx_v, fx_v, fy_v)
            cps[lev] = pltpu.async_copy(tab_hbm.at[idx_v], rows_v, sem)

        for lev in (L - 2, L - 1):
            idx_v, rows_v, fx_v, fy_v, sem = bufs[lev & 1]
            cps[lev].wait()
            combine_fine(lev, rows_v, fx_v, fy_v)

        pltpu.sync_copy(out_v, out_hbm.at[pl.ds(base * (2 * L), C * 2 * L)])


def kernel(xy, tables):
    n = xy.shape[0]
    xs = xy[:, 0]
    ys = xy[:, 1]
    tb = tables.astype(jnp.bfloat16).reshape(L * T, F)
    tabw = lax.bitcast_convert_type(
        lax.bitcast_convert_type(tb, jnp.uint16), jnp.int32
    ).reshape(L * T)
    gidx = jnp.asarray(GIDX)
    mesh = plsc.VectorSubcoreMesh(core_axis_name="c", subcore_axis_name="s")
    run = pl.kernel(
        _body,
        out_type=jax.ShapeDtypeStruct((n * 2 * L,), jnp.float32),
        mesh=mesh,
        compiler_params=pltpu.CompilerParams(needs_layout_passes=False),
        scratch_types=[
            pltpu.VMEM((C,), jnp.float32),
            pltpu.VMEM((C,), jnp.float32),
            pltpu.VMEM((GPAD,), jnp.int32),
            pltpu.VMEM((4 * C,), jnp.int32),
            pltpu.VMEM((4 * C,), jnp.int32),
            pltpu.VMEM((4 * C,), jnp.int32),
            pltpu.VMEM((4 * C,), jnp.int32),
            pltpu.VMEM((C,), jnp.float32),
            pltpu.VMEM((C,), jnp.float32),
            pltpu.VMEM((C,), jnp.float32),
            pltpu.VMEM((C,), jnp.float32),
            pltpu.VMEM((C * 2 * L,), jnp.float32),
            pltpu.SemaphoreType.DMA,
            pltpu.SemaphoreType.DMA,
        ],
    )
    flat = run(xs, ys, tabw, gidx)
    return flat.reshape(n, 2 * L)
